# baseline (speedup 1.0000x reference)
import functools

import jax
import jax.numpy as jnp
import numpy as np
from jax import lax
from jax.experimental import pallas as pl
from jax.experimental.pallas import tpu as pltpu
from jax.experimental.pallas import tpu_sc as plsc

B, L, V, D, NB = 1024, 200, 1000000, 128, 32
N = B * L

C = 4
NCHK = N // C

NC, NS = 2, 16
NW = NC * NS
CHUNK = 80
BUF_ROWS = 160

R4 = 512
BLK = 4 * R4
TC_GRID = N // BLK
TC_GRID_C = TC_GRID // C


def _make_sc_gather(chunk_offset, rows_total, buf_rows):
    rows_per_w = rows_total // NW
    n_groups = rows_per_w // buf_rows
    n_inner = buf_rows // CHUNK
    assert rows_per_w % buf_rows == 0 and n_groups % 2 == 0
    assert buf_rows % CHUNK == 0 and rows_per_w % 8 == 0

    @functools.partial(
        pl.kernel,
        out_type=jax.ShapeDtypeStruct((rows_total, D // 2), jnp.int32),
        mesh=plsc.VectorSubcoreMesh(core_axis_name="c", subcore_axis_name="s"),
        scratch_types=[
            pltpu.VMEM((rows_per_w,), jnp.int32),
            pltpu.VMEM((buf_rows, D), jnp.float32),
            pltpu.VMEM((buf_rows, D), jnp.float32),
            pltpu.VMEM((buf_rows, D // 2), jnp.int32),
            pltpu.VMEM((buf_rows, D // 2), jnp.int32),
            pltpu.SemaphoreType.DMA,
            pltpu.SemaphoreType.DMA,
            pltpu.SemaphoreType.DMA,
            pltpu.SemaphoreType.DMA,
        ],
    )
    def sc_gather(idx_hbm, table_hbm, out_hbm, idx_v, rows0, rows1,
                  bf0, bf1, gsem0, gsem1, wsem0, wsem1):
        wid = lax.axis_index("s") * NC + lax.axis_index("c")
        base = wid * rows_per_w
        pltpu.sync_copy(idx_hbm.at[pl.ds(chunk_offset + base, rows_per_w)],
                        idx_v)
        bufs = (rows0, rows1)
        bfs = (bf0, bf1)
        gsems = (gsem0, gsem1)
        wsems = (wsem0, wsem1)

        def to_bf16_bits(v):
            iv = lax.bitcast_convert_type(v, jnp.int32)
            rnd = iv + 0x7FFF + ((iv >> 16) & 1)
            return (rnd >> 16) & 0xFFFF

        def convert(rows_b, bf_b):
            def conv_row(r, carry):
                for j in range(D // 32):
                    lo = to_bf16_bits(rows_b[r, pl.ds(16 * j, 16)])
                    hi = to_bf16_bits(rows_b[r, pl.ds(64 + 16 * j, 16)])
                    bf_b[r, pl.ds(16 * j, 16)] = lo | (hi << 16)
                return carry
            lax.fori_loop(0, buf_rows, conv_row, 0)

        def fire(g, rows_b, gsem_b):
            row0 = g * buf_rows
            for j in range(n_inner):
                idx_slice = idx_v.at[pl.ds(row0 + j * CHUNK, CHUNK)]
                dst = rows_b.at[pl.ds(j * CHUNK, CHUNK)]
                pltpu.async_copy(table_hbm.at[idx_slice], dst, gsem_b)

        def drain_convert_write(g_prev, rows_p, bf_p, gsem_p, wsem_p):
            pltpu.make_async_copy(
                table_hbm.at[pl.ds(0, buf_rows)], rows_p, gsem_p).wait()
            convert(rows_p, bf_p)
            pltpu.async_copy(
                bf_p, out_hbm.at[pl.ds(base + g_prev * buf_rows, buf_rows)],
                wsem_p)

        def outer(i, carry):
            for b in range(2):
                g = 2 * i + b
                rows_b, bf_b, wsem_b = bufs[b], bfs[b], wsems[b]

                @pl.when(g >= 2)
                def _():
                    pltpu.make_async_copy(
                        bf_b, out_hbm.at[pl.ds(base, buf_rows)], wsem_b
                    ).wait()

                fire(g, rows_b, gsems[b])

                @pl.when(g >= 1)
                def _():
                    drain_convert_write(g - 1, bufs[1 - b], bfs[1 - b],
                                        gsems[1 - b], wsems[1 - b])
            return carry

        lax.fori_loop(0, n_groups // 2, outer, 0)
        drain_convert_write(n_groups - 1, bufs[1], bfs[1], gsems[1], wsems[1])
        for b in range(2):
            pltpu.make_async_copy(
                bfs[b], out_hbm.at[pl.ds(base, buf_rows)], wsems[b]
            ).wait()

    return sc_gather


_sc_gather_chunks = [_make_sc_gather(c * NCHK, NCHK, BUF_ROWS)
                     for c in range(C)]


def _tc_common(afr_ref, tok_ref, freq_ref, w_ref, b_ref, out_ref):
    af4 = jnp.transpose(afr_ref[0], (1, 0))
    lane = lax.broadcasted_iota(jnp.int32, (R4, D), 1) // NB
    afx = jnp.where(
        lane == 0, af4[:, 0:1],
        jnp.where(lane == 1, af4[:, 1:2],
                  jnp.where(lane == 2, af4[:, 2:3], af4[:, 3:4])))
    ang = afx * freq_ref[...]
    w = w_ref[...]
    proj4 = (
        jnp.dot(jnp.sin(ang), w[:D, :], preferred_element_type=jnp.float32)
        + jnp.dot(jnp.cos(ang), w[D:, :], preferred_element_type=jnp.float32)
    )
    H = D // 2
    proj_lo = jnp.concatenate(
        [proj4[:, g * D:g * D + H] for g in range(4)], axis=0)
    proj_hi = jnp.concatenate(
        [proj4[:, g * D + H:(g + 1) * D] for g in range(4)], axis=0)
    t = tok_ref[...]
    lo = lax.bitcast_convert_type(t << 16, jnp.float32)
    hi = lax.bitcast_convert_type(t & jnp.int32(-65536), jnp.float32)
    b = b_ref[...]
    out_ref[:, 0:H] = proj_lo + b[:, 0:H] + lo
    out_ref[:, H:D] = proj_hi + b[:, H:D] + hi


def _tc_body_first(afr_ref, tok_ref, freq_ref, w_ref, b_ref, out_ref):
    _tc_common(afr_ref, tok_ref, freq_ref, w_ref, b_ref, out_ref)


def _tc_body_chained(buf_ref, afr_ref, tok_ref, freq_ref, w_ref, b_ref,
                     out_ref):
    del buf_ref
    _tc_common(afr_ref, tok_ref, freq_ref, w_ref, b_ref, out_ref)


def _tc_embed_chunk(c, buf, af_rows, tok_c, freq, w_all, b2):
    common_specs = [
        pl.BlockSpec((1, 4, 4 * D), lambda i, c=c: (i + c * TC_GRID_C, 0, 0)),
        pl.BlockSpec((BLK, D // 2), lambda i: (i, 0)),
        pl.BlockSpec((1, D), lambda i: (0, 0)),
        pl.BlockSpec((2 * D, 4 * D), lambda i: (0, 0)),
        pl.BlockSpec((1, D), lambda i: (0, 0)),
    ]
    out_spec = pl.BlockSpec((BLK, D), lambda i, c=c: (i + c * TC_GRID_C, 0))
    out_shape = jax.ShapeDtypeStruct((N, D), jnp.float32)
    if buf is None:
        return pl.pallas_call(
            _tc_body_first,
            grid=(TC_GRID_C,),
            in_specs=common_specs,
            out_specs=out_spec,
            out_shape=out_shape,
        )(af_rows, tok_c, freq, w_all, b2)
    return pl.pallas_call(
        _tc_body_chained,
        grid=(TC_GRID_C,),
        in_specs=[pl.BlockSpec(memory_space=pltpu.MemorySpace.HBM)]
        + common_specs,
        out_specs=out_spec,
        out_shape=out_shape,
        input_output_aliases={0: 0},
    )(buf, af_rows, tok_c, freq, w_all, b2)


_FREQ128 = np.tile((2.0 ** np.arange(NB)) * np.pi, 4).astype(np.float32)


def _build_w_all(af_W):
    ws, wc = af_W[:NB], af_W[NB:]
    zero = jnp.zeros((NB, D), jnp.float32)
    def bd(w):
        rows = []
        for g in range(4):
            rows.append(jnp.concatenate(
                [w if gg == g else zero for gg in range(4)], axis=1))
        return jnp.concatenate(rows, axis=0)
    return jnp.concatenate([bd(ws), bd(wc)], axis=0)


@jax.jit
def kernel(seq, af, table, af_W, af_b):
    idx = seq.reshape(N)
    af_rows = af.reshape(TC_GRID, 4, 4 * D)
    freq = jnp.asarray(_FREQ128).reshape(1, D)
    w_all = _build_w_all(af_W)
    b2 = af_b.reshape(1, D)

    toks = [_sc_gather_chunks[c](idx, table) for c in range(C)]
    buf = None
    for c in range(C):
        buf = _tc_embed_chunk(c, buf, af_rows, toks[c], freq, w_all, b2)
    return buf.reshape(B, L, D)


# speedup vs baseline: 1.1569x; 1.1569x over previous
import functools

import jax
import jax.numpy as jnp
import numpy as np
from jax import lax
from jax.experimental import pallas as pl
from jax.experimental.pallas import tpu as pltpu
from jax.experimental.pallas import tpu_sc as plsc

B, L, V, D, NB = 1024, 200, 1000000, 128, 32
N = B * L

C = 5
NCHK = N // C

NC, NS = 2, 16
NW = NC * NS
CHUNK = 80
BUF_ROWS = 160

R4 = 1024
BLK = 4 * R4
TC_GRID = N // BLK
TC_GRID_C = TC_GRID // C


def _make_sc_gather(chunk_offset, rows_total, buf_rows):
    rows_per_w = rows_total // NW
    n_groups = rows_per_w // buf_rows
    n_inner = buf_rows // CHUNK
    assert rows_per_w % buf_rows == 0 and n_groups % 2 == 0
    assert buf_rows % CHUNK == 0 and rows_per_w % 8 == 0

    @functools.partial(
        pl.kernel,
        out_type=jax.ShapeDtypeStruct((rows_total, D), jnp.float32),
        mesh=plsc.VectorSubcoreMesh(core_axis_name="c", subcore_axis_name="s"),
        scratch_types=[
            pltpu.VMEM((rows_per_w,), jnp.int32),
            pltpu.VMEM((buf_rows, D), jnp.float32),
            pltpu.VMEM((buf_rows, D), jnp.float32),
            pltpu.SemaphoreType.DMA,
            pltpu.SemaphoreType.DMA,
            pltpu.SemaphoreType.DMA,
        ],
    )
    def sc_gather(idx_hbm, table_hbm, out_hbm, idx_v, rows0, rows1,
                  gsem, wsem0, wsem1):
        wid = lax.axis_index("s") * NC + lax.axis_index("c")
        base = wid * rows_per_w
        pltpu.sync_copy(idx_hbm.at[pl.ds(chunk_offset + base, rows_per_w)],
                        idx_v)
        bufs = (rows0, rows1)
        wsems = (wsem0, wsem1)

        def outer(i, carry):
            for b in range(2):
                g = 2 * i + b
                row0 = g * buf_rows
                rows_b, wsem_b = bufs[b], wsems[b]

                @pl.when(g >= 2)
                def _():
                    pltpu.make_async_copy(
                        rows_b, out_hbm.at[pl.ds(base, buf_rows)], wsem_b
                    ).wait()

                cps = []
                for j in range(n_inner):
                    idx_slice = idx_v.at[pl.ds(row0 + j * CHUNK, CHUNK)]
                    dst = rows_b.at[pl.ds(j * CHUNK, CHUNK)]
                    cps.append(
                        pltpu.async_copy(table_hbm.at[idx_slice], dst, gsem))
                for cp in cps:
                    cp.wait()
                pltpu.async_copy(
                    rows_b, out_hbm.at[pl.ds(base + row0, buf_rows)], wsem_b)
            return carry

        lax.fori_loop(0, n_groups // 2, outer, 0)
        for b in range(2):
            pltpu.make_async_copy(
                bufs[b], out_hbm.at[pl.ds(base, buf_rows)], wsems[b]
            ).wait()

    return sc_gather


_sc_gather_chunks = [_make_sc_gather(c * NCHK, NCHK, BUF_ROWS)
                     for c in range(C)]


def _tc_common(afr_ref, tok_ref, freq_ref, w_ref, b_ref, out_ref):
    af4 = jnp.transpose(afr_ref[0], (1, 0))
    lane = lax.broadcasted_iota(jnp.int32, (R4, D), 1) // NB
    afx = jnp.where(
        lane == 0, af4[:, 0:1],
        jnp.where(lane == 1, af4[:, 1:2],
                  jnp.where(lane == 2, af4[:, 2:3], af4[:, 3:4])))
    ang = afx * freq_ref[...]
    w = w_ref[...]
    proj4 = (
        jnp.dot(jnp.sin(ang), w[:D, :], preferred_element_type=jnp.float32)
        + jnp.dot(jnp.cos(ang), w[D:, :], preferred_element_type=jnp.float32)
    )
    proj = jnp.concatenate(
        [proj4[:, g * D:(g + 1) * D] for g in range(4)], axis=0)
    out_ref[...] = proj + b_ref[...] + tok_ref[...]


def _tc_body_first(afr_ref, tok_ref, freq_ref, w_ref, b_ref, out_ref):
    _tc_common(afr_ref, tok_ref, freq_ref, w_ref, b_ref, out_ref)


def _tc_body_chained(buf_ref, afr_ref, tok_ref, freq_ref, w_ref, b_ref,
                     out_ref):
    del buf_ref
    _tc_common(afr_ref, tok_ref, freq_ref, w_ref, b_ref, out_ref)


def _tc_embed_chunk(c, buf, af_rows, tok_c, freq, w_all, b2):
    common_specs = [
        pl.BlockSpec((1, 4, R4), lambda i, c=c: (i + c * TC_GRID_C, 0, 0)),
        pl.BlockSpec((BLK, D), lambda i: (i, 0)),
        pl.BlockSpec((1, D), lambda i: (0, 0)),
        pl.BlockSpec((2 * D, 4 * D), lambda i: (0, 0)),
        pl.BlockSpec((1, D), lambda i: (0, 0)),
    ]
    out_spec = pl.BlockSpec((BLK, D), lambda i, c=c: (i + c * TC_GRID_C, 0))
    out_shape = jax.ShapeDtypeStruct((N, D), jnp.float32)
    if buf is None:
        return pl.pallas_call(
            _tc_body_first,
            grid=(TC_GRID_C,),
            in_specs=common_specs,
            out_specs=out_spec,
            out_shape=out_shape,
        )(af_rows, tok_c, freq, w_all, b2)
    return pl.pallas_call(
        _tc_body_chained,
        grid=(TC_GRID_C,),
        in_specs=[pl.BlockSpec(memory_space=pltpu.MemorySpace.HBM)]
        + common_specs,
        out_specs=out_spec,
        out_shape=out_shape,
        input_output_aliases={0: 0},
    )(buf, af_rows, tok_c, freq, w_all, b2)


_FREQ128 = np.tile((2.0 ** np.arange(NB)) * np.pi, 4).astype(np.float32)


def _build_w_all(af_W):
    ws, wc = af_W[:NB], af_W[NB:]
    zero = jnp.zeros((NB, D), jnp.float32)
    def bd(w):
        rows = []
        for g in range(4):
            rows.append(jnp.concatenate(
                [w if gg == g else zero for gg in range(4)], axis=1))
        return jnp.concatenate(rows, axis=0)
    return jnp.concatenate([bd(ws), bd(wc)], axis=0)


@jax.jit
def kernel(seq, af, table, af_W, af_b):
    idx = seq.reshape(N)
    af_rows = af.reshape(TC_GRID, 4, R4)
    freq = jnp.asarray(_FREQ128).reshape(1, D)
    w_all = _build_w_all(af_W)
    b2 = af_b.reshape(1, D)

    toks = [_sc_gather_chunks[c](idx, table) for c in range(C)]
    buf = None
    for c in range(C):
        buf = _tc_embed_chunk(c, buf, af_rows, toks[c], freq, w_all, b2)
    return buf.reshape(B, L, D)


# speedup vs baseline: 1.1935x; 1.0316x over previous
import functools

import jax
import jax.numpy as jnp
import numpy as np
from jax import lax
from jax.experimental import pallas as pl
from jax.experimental.pallas import tpu as pltpu
from jax.experimental.pallas import tpu_sc as plsc

B, L, V, D, NB = 1024, 200, 1000000, 128, 32
N = B * L

C = 5
NCHK = N // C

NC, NS = 2, 16
NW = NC * NS
CHUNK = 80
BUF_ROWS = 160

R4 = 2048
BLK = 4 * R4
TC_GRID = N // BLK
TC_GRID_C = TC_GRID // C


def _make_sc_gather(chunk_offset, rows_total, buf_rows):
    rows_per_w = rows_total // NW
    n_groups = rows_per_w // buf_rows
    n_inner = buf_rows // CHUNK
    assert rows_per_w % buf_rows == 0 and n_groups % 2 == 0
    assert buf_rows % CHUNK == 0 and rows_per_w % 8 == 0

    @functools.partial(
        pl.kernel,
        out_type=jax.ShapeDtypeStruct((rows_total, D), jnp.float32),
        mesh=plsc.VectorSubcoreMesh(core_axis_name="c", subcore_axis_name="s"),
        scratch_types=[
            pltpu.VMEM((rows_per_w,), jnp.int32),
            pltpu.VMEM((buf_rows, D), jnp.float32),
            pltpu.VMEM((buf_rows, D), jnp.float32),
            pltpu.SemaphoreType.DMA,
            pltpu.SemaphoreType.DMA,
            pltpu.SemaphoreType.DMA,
        ],
    )
    def sc_gather(idx_hbm, table_hbm, out_hbm, idx_v, rows0, rows1,
                  gsem, wsem0, wsem1):
        wid = lax.axis_index("s") * NC + lax.axis_index("c")
        base = wid * rows_per_w
        pltpu.sync_copy(idx_hbm.at[pl.ds(chunk_offset + base, rows_per_w)],
                        idx_v)
        bufs = (rows0, rows1)
        wsems = (wsem0, wsem1)

        def outer(i, carry):
            for b in range(2):
                g = 2 * i + b
                row0 = g * buf_rows
                rows_b, wsem_b = bufs[b], wsems[b]

                @pl.when(g >= 2)
                def _():
                    pltpu.make_async_copy(
                        rows_b, out_hbm.at[pl.ds(base, buf_rows)], wsem_b
                    ).wait()

                cps = []
                for j in range(n_inner):
                    idx_slice = idx_v.at[pl.ds(row0 + j * CHUNK, CHUNK)]
                    dst = rows_b.at[pl.ds(j * CHUNK, CHUNK)]
                    cps.append(
                        pltpu.async_copy(table_hbm.at[idx_slice], dst, gsem))
                for cp in cps:
                    cp.wait()
                pltpu.async_copy(
                    rows_b, out_hbm.at[pl.ds(base + row0, buf_rows)], wsem_b)
            return carry

        lax.fori_loop(0, n_groups // 2, outer, 0)
        for b in range(2):
            pltpu.make_async_copy(
                bufs[b], out_hbm.at[pl.ds(base, buf_rows)], wsems[b]
            ).wait()

    return sc_gather


_sc_gather_chunks = [_make_sc_gather(c * NCHK, NCHK, BUF_ROWS)
                     for c in range(C)]


def _tc_common(afr_ref, tok_ref, freq_ref, w_ref, b_ref, out_ref):
    af4 = jnp.transpose(afr_ref[0], (1, 0))
    lane = lax.broadcasted_iota(jnp.int32, (R4, D), 1) // NB
    afx = jnp.where(
        lane == 0, af4[:, 0:1],
        jnp.where(lane == 1, af4[:, 1:2],
                  jnp.where(lane == 2, af4[:, 2:3], af4[:, 3:4])))
    ang = afx * freq_ref[...]
    w = w_ref[...]
    proj4 = (
        jnp.dot(jnp.sin(ang), w[:D, :], preferred_element_type=jnp.float32)
        + jnp.dot(jnp.cos(ang), w[D:, :], preferred_element_type=jnp.float32)
    )
    proj = jnp.concatenate(
        [proj4[:, g * D:(g + 1) * D] for g in range(4)], axis=0)
    out_ref[...] = proj + b_ref[...] + tok_ref[...]


def _tc_body_first(afr_ref, tok_ref, freq_ref, w_ref, b_ref, out_ref):
    _tc_common(afr_ref, tok_ref, freq_ref, w_ref, b_ref, out_ref)


def _tc_body_chained(buf_ref, afr_ref, tok_ref, freq_ref, w_ref, b_ref,
                     out_ref):
    del buf_ref
    _tc_common(afr_ref, tok_ref, freq_ref, w_ref, b_ref, out_ref)


def _tc_embed_chunk(c, buf, af_rows, tok_c, freq, w_all, b2):
    common_specs = [
        pl.BlockSpec((1, 4, R4), lambda i, c=c: (i + c * TC_GRID_C, 0, 0)),
        pl.BlockSpec((BLK, D), lambda i: (i, 0)),
        pl.BlockSpec((1, D), lambda i: (0, 0)),
        pl.BlockSpec((2 * D, 4 * D), lambda i: (0, 0)),
        pl.BlockSpec((1, D), lambda i: (0, 0)),
    ]
    out_spec = pl.BlockSpec((BLK, D), lambda i, c=c: (i + c * TC_GRID_C, 0))
    out_shape = jax.ShapeDtypeStruct((N, D), jnp.float32)
    if buf is None:
        return pl.pallas_call(
            _tc_body_first,
            grid=(TC_GRID_C,),
            in_specs=common_specs,
            out_specs=out_spec,
            out_shape=out_shape,
        )(af_rows, tok_c, freq, w_all, b2)
    return pl.pallas_call(
        _tc_body_chained,
        grid=(TC_GRID_C,),
        in_specs=[pl.BlockSpec(memory_space=pltpu.MemorySpace.HBM)]
        + common_specs,
        out_specs=out_spec,
        out_shape=out_shape,
        input_output_aliases={0: 0},
    )(buf, af_rows, tok_c, freq, w_all, b2)


_FREQ128 = np.tile((2.0 ** np.arange(NB)) * np.pi, 4).astype(np.float32)


def _build_w_all(af_W):
    ws, wc = af_W[:NB], af_W[NB:]
    zero = jnp.zeros((NB, D), jnp.float32)
    def bd(w):
        rows = []
        for g in range(4):
            rows.append(jnp.concatenate(
                [w if gg == g else zero for gg in range(4)], axis=1))
        return jnp.concatenate(rows, axis=0)
    return jnp.concatenate([bd(ws), bd(wc)], axis=0)


@jax.jit
def kernel(seq, af, table, af_W, af_b):
    idx = seq.reshape(N)
    af_rows = af.reshape(TC_GRID, 4, R4)
    freq = jnp.asarray(_FREQ128).reshape(1, D)
    w_all = _build_w_all(af_W)
    b2 = af_b.reshape(1, D)

    toks = [_sc_gather_chunks[c](idx, table) for c in range(C)]
    buf = None
    for c in range(C):
        buf = _tc_embed_chunk(c, buf, af_rows, toks[c], freq, w_all, b2)
    return buf.reshape(B, L, D)
